# baseline (speedup 1.0000x reference)
import functools

import jax
import jax.numpy as jnp
from jax import lax
from jax.experimental import pallas as pl
from jax.experimental.pallas import tpu as pltpu
from jax.experimental.pallas import tpu_sc as plsc

VOCAB_SIZE = 1000000
EMBED_DIM = 64
SEQ_LEN = 200
BATCH = 1024

NUM_WORKERS = 32
BLK = 128
NBLK = BATCH // BLK
UNITS = SEQ_LEN * NBLK
UNITS_PER_WORKER = UNITS // NUM_WORKERS
LANES = 16
GROUPS = BLK // LANES


def _body(xT_hbm, tok_hbm, pos_hbm, out_hbm,
          idx_v, rid_v, g_v, m_v, pos_v, sem):
    wid = lax.axis_index("s") * 2 + lax.axis_index("c")
    t0 = wid * UNITS_PER_WORKER

    pltpu.sync_copy(pos_hbm, pos_v)

    def run_unit(t, carry):
        s = t // NBLK
        v = t % NBLK

        pltpu.sync_copy(xT_hbm.at[s, pl.ds(v * BLK, BLK)], idx_v)

        def shift_body(i, c):
            sl = pl.ds(i * LANES, LANES)
            rid_v[sl] = lax.shift_right_logical(idx_v[sl], 1)
            return c

        lax.fori_loop(0, GROUPS, shift_body, 0, unroll=True)

        pltpu.async_copy(tok_hbm.at[rid_v], g_v, sem).wait()

        def col_group(c0, carry2):
            row_idx = lax.iota(jnp.int32, LANES) + c0 * LANES
            col_base = (idx_v[pl.ds(c0 * LANES, LANES)] & 1) * EMBED_DIM

            def a_body(a, carry3):
                pvec = plsc.load_gather(
                    pos_v, [jnp.broadcast_to(s * EMBED_DIM + a, (LANES,))])
                vals = plsc.load_gather(g_v, [row_idx, col_base + a])
                m_v[a, pl.ds(c0 * LANES, LANES)] = vals + pvec
                return carry3

            lax.fori_loop(0, EMBED_DIM, a_body, 0)
            return carry2

        lax.fori_loop(0, GROUPS, col_group, 0, unroll=True)

        pltpu.sync_copy(m_v, out_hbm.at[s, :, pl.ds(v * BLK, BLK)])
        return carry

    lax.fori_loop(t0, t0 + UNITS_PER_WORKER, run_unit, 0)


def kernel(x, token_table, pos_table):
    xT = x.T.astype(jnp.int32)
    tok2 = token_table.reshape(VOCAB_SIZE // 2, 2 * EMBED_DIM)
    pos_flat = pos_table.reshape(SEQ_LEN * EMBED_DIM)
    mesh = plsc.VectorSubcoreMesh(core_axis_name="c", subcore_axis_name="s")
    run = functools.partial(
        pl.kernel,
        mesh=mesh,
        out_type=jax.ShapeDtypeStruct((SEQ_LEN, EMBED_DIM, BATCH),
                                      jnp.float32),
        scratch_types=[
            pltpu.VMEM((BLK,), jnp.int32),
            pltpu.VMEM((BLK,), jnp.int32),
            pltpu.VMEM((BLK, 2 * EMBED_DIM), jnp.float32),
            pltpu.VMEM((EMBED_DIM, BLK), jnp.float32),
            pltpu.VMEM((SEQ_LEN * EMBED_DIM,), jnp.float32),
            pltpu.SemaphoreType.DMA,
        ],
        compiler_params=pltpu.CompilerParams(needs_layout_passes=False),
    )(_body)
    out = run(xT, tok2, pos_flat)
    return out.transpose(2, 0, 1)
